# baseline (speedup 1.0000x reference)
import functools

import jax
import jax.numpy as jnp
from jax.experimental import pallas as pl
from jax.experimental.pallas import tpu as pltpu

H, W, C_IN, C_OUT = 181, 360, 96, 48
NG = H * W
NM = 10242
E_G2M = 2 * NG
E_M = 8 * NM
E_M2G = 2 * NG
D = 128
L = 4

BR = 256
NGP = 65280
NMP = 10496
EGP = 131072
EMP = 86016

_F32 = jnp.float32


def _rows_bs(ncols):
    return pl.BlockSpec((BR, ncols), lambda i: (i, 0))


def _full_bs(shape):
    nd = len(shape)
    return pl.BlockSpec(shape, lambda i: (0,) * nd)


def _ln(h, g, bn):
    mu = jnp.mean(h, axis=-1, keepdims=True)
    var = jnp.mean((h - mu) ** 2, axis=-1, keepdims=True)
    return (h - mu) * jax.lax.rsqrt(var + 1e-5) * g + bn


def _silu(x):
    return x * jax.nn.sigmoid(x)


def _enc_wide(x, W0, b0, W1, b1, g, bn, tables):
    n = x.shape[0]
    nt = len(tables)

    def body(x_r, W0_r, b0_r, W1_r, b1_r, g_r, bn_r, *rest):
        t_refs = rest[:nt]
        o_refs = rest[nt:]
        h = _silu(jnp.dot(x_r[...], W0_r[...], preferred_element_type=_F32)
                  + b0_r[...])
        y = _ln(jnp.dot(h, W1_r[...], preferred_element_type=_F32) + b1_r[...],
                g_r[...], bn_r[...])
        o_refs[0][...] = y
        for k in range(nt):
            o_refs[k + 1][...] = jnp.dot(y, t_refs[k][...],
                                         preferred_element_type=_F32)

    out_shape = [jax.ShapeDtypeStruct((n, D), _F32)] * (1 + nt)
    return pl.pallas_call(
        body,
        grid=(n // BR,),
        in_specs=[_rows_bs(x.shape[1]), _full_bs(W0.shape), _full_bs((1, D)),
                  _full_bs((D, D)), _full_bs((1, D)), _full_bs((1, D)),
                  _full_bs((1, D))] + [_full_bs((D, D))] * nt,
        out_specs=[_rows_bs(D)] * (1 + nt),
        out_shape=out_shape,
    )(x, W0, b0.reshape(1, D), W1, b1.reshape(1, D), g.reshape(1, D),
      bn.reshape(1, D), *tables)


def _enc_cols(x, W0, b0, W1, b1, g, bn, tables, emit_y=True):
    n, c = x.shape
    nt = len(tables)

    def body(x_r, W0_r, b0_r, W1_r, b1_r, g_r, bn_r, *rest):
        t_refs = rest[:nt]
        o_refs = rest[nt:]
        acc = b0_r[...]
        xv = x_r[...]
        w0 = W0_r[...]
        for k in range(c):
            acc = acc + xv[:, k:k + 1] * w0[k:k + 1, :]
        h = _silu(acc)
        y = _ln(jnp.dot(h, W1_r[...], preferred_element_type=_F32) + b1_r[...],
                g_r[...], bn_r[...])
        o = 0
        if emit_y:
            o_refs[0][...] = y
            o = 1
        for k in range(nt):
            o_refs[o + k][...] = jnp.dot(y, t_refs[k][...],
                                         preferred_element_type=_F32)

    n_out = (1 if emit_y else 0) + nt
    return pl.pallas_call(
        body,
        grid=(n // BR,),
        in_specs=[_rows_bs(c), _full_bs((c, D)), _full_bs((1, D)),
                  _full_bs((D, D)), _full_bs((1, D)), _full_bs((1, D)),
                  _full_bs((1, D))] + [_full_bs((D, D))] * nt,
        out_specs=[_rows_bs(D)] * n_out,
        out_shape=[jax.ShapeDtypeStruct((n, D), _F32)] * n_out,
    )(x, W0, b0.reshape(1, D), W1, b1.reshape(1, D), g.reshape(1, D),
      bn.reshape(1, D), *tables)


def _edge_mlp(eterm, gs, gd, b0, W1, b1, g, bn, e_in=None, Wnext=None):
    n = eterm.shape[0]
    has_state = e_in is not None
    has_next = Wnext is not None

    def body(*refs):
        i = 0
        eterm_r = refs[i]; i += 1
        gs_r = refs[i]; i += 1
        gd_r = refs[i]; i += 1
        b0_r = refs[i]; i += 1
        W1_r = refs[i]; i += 1
        b1_r = refs[i]; i += 1
        g_r = refs[i]; i += 1
        bn_r = refs[i]; i += 1
        e_r = None
        Wn_r = None
        if has_state:
            e_r = refs[i]; i += 1
        if has_next:
            Wn_r = refs[i]; i += 1
        outs = refs[i:]
        h = _silu(eterm_r[...] + gs_r[...] + gd_r[...] + b0_r[...])
        eu = _ln(jnp.dot(h, W1_r[...], preferred_element_type=_F32)
                 + b1_r[...], g_r[...], bn_r[...])
        outs[0][...] = eu
        o = 1
        if has_state:
            e_new = e_r[...] + eu
            outs[o][...] = e_new
            o += 1
            if has_next:
                outs[o][...] = jnp.dot(e_new, Wn_r[...],
                                       preferred_element_type=_F32)

    n_out = 1 + (1 if has_state else 0) + (1 if (has_state and has_next) else 0)
    in_arrs = [eterm, gs, gd, b0.reshape(1, D), W1, b1.reshape(1, D),
               g.reshape(1, D), bn.reshape(1, D)]
    in_specs = [_rows_bs(D), _rows_bs(D), _rows_bs(D), _full_bs((1, D)),
                _full_bs((D, D)), _full_bs((1, D)), _full_bs((1, D)),
                _full_bs((1, D))]
    if has_state:
        in_arrs.append(e_in)
        in_specs.append(_rows_bs(D))
    if has_next:
        in_arrs.append(Wnext)
        in_specs.append(_full_bs((D, D)))
    return pl.pallas_call(
        body,
        grid=(n // BR,),
        in_specs=in_specs,
        out_specs=[_rows_bs(D)] * n_out,
        out_shape=[jax.ShapeDtypeStruct((n, D), _F32)] * n_out,
    )(*in_arrs)


def _node_mlp(x, parts, W0x, W0a, b0, W1, b1, g, bn, tables=(), dec=None):
    n = x.shape[0]
    np_ = len(parts)
    nt = len(tables)
    has_dec = dec is not None

    def body(*refs):
        i = 0
        x_r = refs[i]; i += 1
        p_refs = refs[i:i + np_]; i += np_
        W0x_r = refs[i]; i += 1
        W0a_r = refs[i]; i += 1
        b0_r = refs[i]; i += 1
        W1_r = refs[i]; i += 1
        b1_r = refs[i]; i += 1
        g_r = refs[i]; i += 1
        bn_r = refs[i]; i += 1
        t_refs = refs[i:i + nt]; i += nt
        if has_dec:
            Wd0_r = refs[i]; i += 1
            bd0_r = refs[i]; i += 1
            Wd1_r = refs[i]; i += 1
            bd1_r = refs[i]; i += 1
        outs = refs[i:]
        agg = p_refs[0][...]
        for k in range(1, np_):
            agg = agg + p_refs[k][...]
        xv = x_r[...]
        h = _silu(jnp.dot(xv, W0x_r[...], preferred_element_type=_F32)
                  + jnp.dot(agg, W0a_r[...], preferred_element_type=_F32)
                  + b0_r[...])
        y = _ln(jnp.dot(h, W1_r[...], preferred_element_type=_F32)
                + b1_r[...], g_r[...], bn_r[...])
        x_new = xv + y
        o = 0
        if has_dec:
            hd = _silu(jnp.dot(x_new, Wd0_r[...],
                               preferred_element_type=_F32) + bd0_r[...])
            outs[0][...] = jnp.dot(hd, Wd1_r[...],
                                   preferred_element_type=_F32) + bd1_r[...]
            o = 1
        else:
            outs[0][...] = x_new
            o = 1
        for k in range(nt):
            outs[o + k][...] = jnp.dot(x_new, t_refs[k][...],
                                       preferred_element_type=_F32)

    in_arrs = [x] + list(parts) + [W0x, W0a, b0.reshape(1, D), W1,
                                   b1.reshape(1, D), g.reshape(1, D),
                                   bn.reshape(1, D)] + list(tables)
    in_specs = ([_rows_bs(D)] * (1 + np_)
                + [_full_bs((D, D)), _full_bs((D, D)), _full_bs((1, D)),
                   _full_bs((D, D)), _full_bs((1, D)), _full_bs((1, D)),
                   _full_bs((1, D))] + [_full_bs((D, D))] * nt)
    if has_dec:
        Wd0, bd0, Wd1, bd1 = dec
        in_arrs += [Wd0, bd0.reshape(1, D), Wd1, bd1.reshape(1, D)]
        in_specs += [_full_bs((D, D)), _full_bs((1, D)), _full_bs((D, D)),
                     _full_bs((1, D))]
    n_out = 1 + nt
    return pl.pallas_call(
        body,
        grid=(n // BR,),
        in_specs=in_specs,
        out_specs=[_rows_bs(D)] * n_out,
        out_shape=[jax.ShapeDtypeStruct((n, D), _F32)] * n_out,
    )(*in_arrs)


def _gather(table, idx):
    return jnp.take(table, idx, axis=0, mode='clip')


def _segsum(vals, dst, nseg):
    return jax.ops.segment_sum(vals, dst, num_segments=nseg)


def _pad_rows(x, n):
    return jnp.pad(x, ((0, n - x.shape[0]), (0, 0)))


def _pad_idx(idx, n):
    return jnp.pad(idx.astype(jnp.int32), (0, n - idx.shape[0]))


def kernel(grid_nfeat, mesh_nfeat, g2m_efeat, mesh_efeat, m2g_efeat,
           g2m_src, g2m_dst, mesh_src, mesh_dst, m2g_src, m2g_dst, params):
    p = params

    def W0(q):
        return q['l0']['W']

    def mlpw(q):
        return q['l0']['W'], q['l0']['b'], q['l1']['W'], q['l1']['b']

    def mlpn(q):
        return q['l0']['b'], q['l1']['W'], q['l1']['b'], q['g'], q['bn']

    x_grid = grid_nfeat[0].reshape(C_IN, NG).transpose(1, 0)
    x_grid = jnp.pad(x_grid, ((0, NGP - NG), (0, D - C_IN)))
    mesh_n = _pad_rows(mesh_nfeat, NMP)
    ge = _pad_rows(g2m_efeat, EGP)
    me = _pad_rows(mesh_efeat, EMP)
    de = _pad_rows(m2g_efeat, EGP)

    g2m_src_p = _pad_idx(g2m_src, EGP)
    m2g_src_p = _pad_idx(m2g_src, EGP)
    mesh_src_p = _pad_idx(mesh_src, EMP)
    g2m_dst_p = jnp.pad(g2m_dst.astype(jnp.int32), (0, EGP - E_G2M),
                        constant_values=NM)
    mesh_dst_p = jnp.pad(mesh_dst.astype(jnp.int32), (0, EMP - E_M),
                         constant_values=NM)
    m2g_dst_p = jnp.pad(m2g_dst.astype(jnp.int32), (0, EGP - E_M2G),
                        constant_values=NG)

    w_g2m = W0(p['g2m_edge_mlp'])
    w_g2m_e, w_g2m_s, w_g2m_d = w_g2m[:D], w_g2m[D:2 * D], w_g2m[2 * D:]
    w_m2g = W0(p['m2g_edge_mlp'])
    w_m2g_e, w_m2g_s, w_m2g_d = w_m2g[:D], w_m2g[D:2 * D], w_m2g[2 * D:]
    w_pe = [W0(p['proc%d_edge' % i]) for i in range(L)]
    w_pe_e = [w[:D] for w in w_pe]
    w_pe_s = [w[D:2 * D] for w in w_pe]
    w_pe_d = [w[2 * D:] for w in w_pe]

    gw0, gb0, gw1, gb1 = mlpw(p['grid_enc'])
    gw0 = jnp.pad(gw0, ((0, D - C_IN), (0, 0)))
    gfeat, t_g2m_src = _enc_wide(x_grid, gw0, gb0, gw1, gb1,
                                 p['grid_enc']['g'], p['grid_enc']['bn'],
                                 [w_g2m_s])
    mfeat, t_g2m_dst = _enc_cols(mesh_n, *mlpw(p['mesh_enc'])[0:1],
                                 *mlpn(p['mesh_enc']), tables=[w_g2m_d])
    (eterm_g2m,) = _enc_cols(ge, W0(p['g2m_edge_enc']),
                             *mlpn(p['g2m_edge_enc']), tables=[w_g2m_e],
                             emit_y=False)

    gs = _gather(t_g2m_src, g2m_src_p)
    gd = _gather(t_g2m_dst, g2m_dst_p)
    q = p['g2m_edge_mlp']
    (eu,) = _edge_mlp(eterm_g2m, gs, gd, *mlpn(q))
    agg = _segsum(eu[:E_G2M], g2m_dst_p[:E_G2M], NMP)
    q = p['g2m_node_mlp']
    w0q = W0(q)
    mfeat, ts0, td0 = _node_mlp(mfeat, [agg], w0q[:D], w0q[D:], *mlpn(q),
                                tables=[w_pe_s[0], w_pe_d[0]])
    q = p['g2m_grid_mlp']
    gfeat, t_m2g_dst = _enc_wide_residual(gfeat, W0(q), *mlpn(q), [w_m2g_d])

    q = p['mesh_edge_enc']
    e_m, eterm = _enc_cols(me, W0(q), *mlpn(q), tables=[w_pe_e[0]],
                           emit_y=True)
    ts, td = ts0, td0
    for i in range(L):
        gs = _gather(ts, mesh_src_p)
        gd = _gather(td, mesh_dst_p)
        q = p['proc%d_edge' % i]
        if i < L - 1:
            eu, e_m, eterm = _edge_mlp(eterm, gs, gd, *mlpn(q), e_in=e_m,
                                       Wnext=w_pe_e[i + 1])
        else:
            (eu,) = _edge_mlp(eterm, gs, gd, *mlpn(q))
        agg = _segsum(eu[:E_M], mesh_dst_p[:E_M], NMP)
        q = p['proc%d_node' % i]
        w0q = W0(q)
        if i < L - 1:
            tabs = [w_pe_s[i + 1], w_pe_d[i + 1]]
        else:
            tabs = [w_m2g_s]
        outs = _node_mlp(mfeat, [agg], w0q[:D], w0q[D:], *mlpn(q),
                         tables=tabs)
        mfeat = outs[0]
        if i < L - 1:
            ts, td = outs[1], outs[2]
        else:
            t_m2g_src = outs[1]

    q = p['m2g_edge_enc']
    (eterm_m2g,) = _enc_cols(de, W0(q), *mlpn(q), tables=[w_m2g_e],
                             emit_y=False)
    gs = _gather(t_m2g_src, m2g_src_p)
    gd = _gather(t_m2g_dst, m2g_dst_p)
    q = p['m2g_edge_mlp']
    (eu,) = _edge_mlp(eterm_m2g, gs, gd, *mlpn(q))
    agg = _segsum(eu[:E_M2G], m2g_dst_p[:E_M2G], NGP)
    q = p['m2g_node_mlp']
    w0q = W0(q)
    dq = p['dec']
    dw1 = jnp.pad(dq['l1']['W'], ((0, 0), (0, D - C_OUT)))
    db1 = jnp.pad(dq['l1']['b'], (0, D - C_OUT))
    (out128,) = _node_mlp(gfeat, [agg], w0q[:D], w0q[D:], *mlpn(q),
                          dec=(dq['l0']['W'], dq['l0']['b'], dw1, db1))

    out = out128[:NG, :C_OUT]
    return out.transpose(1, 0).reshape(1, C_OUT, H, W)


def _enc_wide_residual(x, W0, b0, W1, b1, g, bn, tables):
    n = x.shape[0]
    nt = len(tables)

    def body(x_r, W0_r, b0_r, W1_r, b1_r, g_r, bn_r, *rest):
        t_refs = rest[:nt]
        o_refs = rest[nt:]
        xv = x_r[...]
        h = _silu(jnp.dot(xv, W0_r[...], preferred_element_type=_F32)
                  + b0_r[...])
        y = xv + _ln(jnp.dot(h, W1_r[...], preferred_element_type=_F32)
                     + b1_r[...], g_r[...], bn_r[...])
        o_refs[0][...] = y
        for k in range(nt):
            o_refs[k + 1][...] = jnp.dot(y, t_refs[k][...],
                                         preferred_element_type=_F32)

    return pl.pallas_call(
        body,
        grid=(n // BR,),
        in_specs=[_rows_bs(D), _full_bs((D, D)), _full_bs((1, D)),
                  _full_bs((D, D)), _full_bs((1, D)), _full_bs((1, D)),
                  _full_bs((1, D))] + [_full_bs((D, D))] * nt,
        out_specs=[_rows_bs(D)] * (1 + nt),
        out_shape=[jax.ShapeDtypeStruct((n, D), _F32)] * (1 + nt),
    )(x, W0, b0.reshape(1, D), W1, b1.reshape(1, D), g.reshape(1, D),
      bn.reshape(1, D), *tables)


# speedup vs baseline: 2.0707x; 2.0707x over previous
import functools

import jax
import jax.numpy as jnp
from jax import lax
from jax.experimental import pallas as pl
from jax.experimental.pallas import tpu as pltpu
from jax.experimental.pallas import tpu_sc as plsc

H, W, C_IN, C_OUT = 181, 360, 96, 48
NG = H * W
NM = 10242
E_G2M = 2 * NG
E_M = 8 * NM
E_M2G = 2 * NG
D = 128
L = 4

BR = 512
NGP = 65536
NMP = 10752
EGP = 131072
EMP = 86016

NW = 32
BB = 128

_F32 = jnp.float32


def _rows_bs(ncols):
    return pl.BlockSpec((BR, ncols), lambda i: (i, 0))


def _full_bs(shape):
    nd = len(shape)
    return pl.BlockSpec(shape, lambda i: (0,) * nd)


def _ln(h, g, bn):
    mu = jnp.mean(h, axis=-1, keepdims=True)
    var = jnp.mean((h - mu) ** 2, axis=-1, keepdims=True)
    return (h - mu) * jax.lax.rsqrt(var + 1e-5) * g + bn


def _silu(x):
    return x * jax.nn.sigmoid(x)


def _enc_wide(x, W0, b0, W1, b1, g, bn, tables):
    n = x.shape[0]
    nt = len(tables)

    def body(x_r, W0_r, b0_r, W1_r, b1_r, g_r, bn_r, *rest):
        t_refs = rest[:nt]
        o_refs = rest[nt:]
        h = _silu(jnp.dot(x_r[...], W0_r[...], preferred_element_type=_F32)
                  + b0_r[...])
        y = _ln(jnp.dot(h, W1_r[...], preferred_element_type=_F32) + b1_r[...],
                g_r[...], bn_r[...])
        o_refs[0][...] = y
        for k in range(nt):
            o_refs[k + 1][...] = jnp.dot(y, t_refs[k][...],
                                         preferred_element_type=_F32)

    out_shape = [jax.ShapeDtypeStruct((n, D), _F32)] * (1 + nt)
    return pl.pallas_call(
        body,
        grid=(n // BR,),
        in_specs=[_rows_bs(x.shape[1]), _full_bs(W0.shape), _full_bs((1, D)),
                  _full_bs((D, D)), _full_bs((1, D)), _full_bs((1, D)),
                  _full_bs((1, D))] + [_full_bs((D, D))] * nt,
        out_specs=[_rows_bs(D)] * (1 + nt),
        out_shape=out_shape,
    )(x, W0, b0.reshape(1, D), W1, b1.reshape(1, D), g.reshape(1, D),
      bn.reshape(1, D), *tables)


def _enc_cols(x, W0, b0, W1, b1, g, bn, tables, emit_y=True):
    n, c = x.shape
    nt = len(tables)

    def body(x_r, W0_r, b0_r, W1_r, b1_r, g_r, bn_r, *rest):
        t_refs = rest[:nt]
        o_refs = rest[nt:]
        acc = b0_r[...]
        xv = x_r[...]
        w0 = W0_r[...]
        for k in range(c):
            acc = acc + xv[:, k:k + 1] * w0[k:k + 1, :]
        h = _silu(acc)
        y = _ln(jnp.dot(h, W1_r[...], preferred_element_type=_F32) + b1_r[...],
                g_r[...], bn_r[...])
        o = 0
        if emit_y:
            o_refs[0][...] = y
            o = 1
        for k in range(nt):
            o_refs[o + k][...] = jnp.dot(y, t_refs[k][...],
                                         preferred_element_type=_F32)

    n_out = (1 if emit_y else 0) + nt
    return pl.pallas_call(
        body,
        grid=(n // BR,),
        in_specs=[_rows_bs(c), _full_bs((c, D)), _full_bs((1, D)),
                  _full_bs((D, D)), _full_bs((1, D)), _full_bs((1, D)),
                  _full_bs((1, D))] + [_full_bs((D, D))] * nt,
        out_specs=[_rows_bs(D)] * n_out,
        out_shape=[jax.ShapeDtypeStruct((n, D), _F32)] * n_out,
    )(x, W0, b0.reshape(1, D), W1, b1.reshape(1, D), g.reshape(1, D),
      bn.reshape(1, D), *tables)


def _edge_mlp(eterm, gs, gd, b0, W1, b1, g, bn, e_in=None, Wnext=None):
    n = eterm.shape[0]
    has_state = e_in is not None
    has_next = Wnext is not None

    def body(*refs):
        i = 0
        eterm_r = refs[i]; i += 1
        gs_r = refs[i]; i += 1
        gd_r = refs[i]; i += 1
        b0_r = refs[i]; i += 1
        W1_r = refs[i]; i += 1
        b1_r = refs[i]; i += 1
        g_r = refs[i]; i += 1
        bn_r = refs[i]; i += 1
        e_r = None
        Wn_r = None
        if has_state:
            e_r = refs[i]; i += 1
        if has_next:
            Wn_r = refs[i]; i += 1
        outs = refs[i:]
        h = _silu(eterm_r[...] + gs_r[...] + gd_r[...] + b0_r[...])
        eu = _ln(jnp.dot(h, W1_r[...], preferred_element_type=_F32)
                 + b1_r[...], g_r[...], bn_r[...])
        outs[0][...] = eu
        o = 1
        if has_state:
            e_new = e_r[...] + eu
            outs[o][...] = e_new
            o += 1
            if has_next:
                outs[o][...] = jnp.dot(e_new, Wn_r[...],
                                       preferred_element_type=_F32)

    n_out = 1 + (1 if has_state else 0) + (1 if (has_state and has_next) else 0)
    in_arrs = [eterm, gs, gd, b0.reshape(1, D), W1, b1.reshape(1, D),
               g.reshape(1, D), bn.reshape(1, D)]
    in_specs = [_rows_bs(D), _rows_bs(D), _rows_bs(D), _full_bs((1, D)),
                _full_bs((D, D)), _full_bs((1, D)), _full_bs((1, D)),
                _full_bs((1, D))]
    if has_state:
        in_arrs.append(e_in)
        in_specs.append(_rows_bs(D))
    if has_next:
        in_arrs.append(Wnext)
        in_specs.append(_full_bs((D, D)))
    return pl.pallas_call(
        body,
        grid=(n // BR,),
        in_specs=in_specs,
        out_specs=[_rows_bs(D)] * n_out,
        out_shape=[jax.ShapeDtypeStruct((n, D), _F32)] * n_out,
    )(*in_arrs)


def _node_mlp(x, aggs, W0x, W0a, b0, W1, b1, g, bn, tables=(), dec=None):
    n = x.shape[0]
    ns = aggs.shape[0]
    nt = len(tables)
    has_dec = dec is not None

    def body(*refs):
        i = 0
        x_r = refs[i]; i += 1
        a_r = refs[i]; i += 1
        W0x_r = refs[i]; i += 1
        W0a_r = refs[i]; i += 1
        b0_r = refs[i]; i += 1
        W1_r = refs[i]; i += 1
        b1_r = refs[i]; i += 1
        g_r = refs[i]; i += 1
        bn_r = refs[i]; i += 1
        t_refs = refs[i:i + nt]; i += nt
        if has_dec:
            Wd0_r = refs[i]; i += 1
            bd0_r = refs[i]; i += 1
            Wd1_r = refs[i]; i += 1
            bd1_r = refs[i]; i += 1
        outs = refs[i:]
        av = a_r[...]
        agg = av[0]
        for k in range(1, ns):
            agg = agg + av[k]
        xv = x_r[...]
        h = _silu(jnp.dot(xv, W0x_r[...], preferred_element_type=_F32)
                  + jnp.dot(agg, W0a_r[...], preferred_element_type=_F32)
                  + b0_r[...])
        y = _ln(jnp.dot(h, W1_r[...], preferred_element_type=_F32)
                + b1_r[...], g_r[...], bn_r[...])
        x_new = xv + y
        o = 0
        if has_dec:
            hd = _silu(jnp.dot(x_new, Wd0_r[...],
                               preferred_element_type=_F32) + bd0_r[...])
            outs[0][...] = jnp.dot(hd, Wd1_r[...],
                                   preferred_element_type=_F32) + bd1_r[...]
            o = 1
        else:
            outs[0][...] = x_new
            o = 1
        for k in range(nt):
            outs[o + k][...] = jnp.dot(x_new, t_refs[k][...],
                                       preferred_element_type=_F32)

    in_arrs = [x, aggs] + [W0x, W0a, b0.reshape(1, D), W1,
                           b1.reshape(1, D), g.reshape(1, D),
                           bn.reshape(1, D)] + list(tables)
    in_specs = ([_rows_bs(D),
                 pl.BlockSpec((ns, BR, D), lambda i: (0, i, 0))]
                + [_full_bs((D, D)), _full_bs((D, D)), _full_bs((1, D)),
                   _full_bs((D, D)), _full_bs((1, D)), _full_bs((1, D)),
                   _full_bs((1, D))] + [_full_bs((D, D))] * nt)
    if has_dec:
        Wd0, bd0, Wd1, bd1 = dec
        in_arrs += [Wd0, bd0.reshape(1, D), Wd1, bd1.reshape(1, D)]
        in_specs += [_full_bs((D, D)), _full_bs((1, D)), _full_bs((D, D)),
                     _full_bs((1, D))]
    n_out = 1 + nt
    return pl.pallas_call(
        body,
        grid=(n // BR,),
        in_specs=in_specs,
        out_specs=[_rows_bs(D)] * n_out,
        out_shape=[jax.ShapeDtypeStruct((n, D), _F32)] * n_out,
    )(*in_arrs)


def _sc_gather2(tabA, idxA, tabB, idxB):
    EP = idxA.shape[0]
    KB = EP // NW
    K = KB // BB
    mesh = plsc.VectorSubcoreMesh(core_axis_name="c", subcore_axis_name="s")

    @functools.partial(
        pl.kernel,
        out_type=[jax.ShapeDtypeStruct((EP, D), jnp.float32),
                  jax.ShapeDtypeStruct((EP, D), jnp.float32)],
        mesh=mesh,
        scratch_types=[pltpu.VMEM((KB,), jnp.int32),
                       pltpu.VMEM((2, BB, D), jnp.float32),
                       pltpu.SemaphoreType.DMA, pltpu.SemaphoreType.DMA,
                       pltpu.SemaphoreType.DMA, pltpu.SemaphoreType.DMA],
    )
    def k(tabA_h, idxA_h, tabB_h, idxB_h, outA_h, outB_h,
          idx_v, rows_v, g0, g1, w0, w1):
        wid = lax.axis_index("s") * 2 + lax.axis_index("c")
        base = pl.multiple_of(wid * KB, BB)
        gsems = (g0, g1)
        wsems = (w0, w1)
        for tab_h, idx_h, out_h in ((tabA_h, idxA_h, outA_h),
                                    (tabB_h, idxB_h, outB_h)):
            pltpu.sync_copy(idx_h.at[pl.ds(base, KB)], idx_v)
            gd = [None] * K
            wd = [None] * K
            for j in range(K):
                b = j % 2
                if j >= 2:
                    wd[j - 2].wait()
                gd[j] = pltpu.async_copy(
                    tab_h.at[idx_v.at[pl.ds(j * BB, BB)]], rows_v.at[b],
                    gsems[b])
                if j >= 1:
                    gd[j - 1].wait()
                    wd[j - 1] = pltpu.async_copy(
                        rows_v.at[(j - 1) % 2],
                        out_h.at[pl.ds(base + (j - 1) * BB, BB)],
                        wsems[(j - 1) % 2])
            gd[K - 1].wait()
            wd[K - 1] = pltpu.async_copy(
                rows_v.at[(K - 1) % 2],
                out_h.at[pl.ds(base + (K - 1) * BB, BB)], wsems[(K - 1) % 2])
            wd[K - 2].wait()
            wd[K - 1].wait()

    return k(tabA, idxA, tabB, idxB)


def _sc_segsum(eu, idx2d, nseg, npass, chunk_b):
    EP = eu.shape[0]
    KB = EP // NW
    K = KB // BB
    NCH = KB // (chunk_b * BB)
    CW = D // npass
    SR = nseg // 16
    CR = chunk_b * BB
    mesh = plsc.VectorSubcoreMesh(core_axis_name="c", subcore_axis_name="s")
    zeros = jnp.zeros((nseg, CW), jnp.float32)

    @functools.partial(
        pl.kernel,
        out_type=jax.ShapeDtypeStruct((2, nseg, D), jnp.float32),
        mesh=mesh,
        scratch_types=[pltpu.VMEM((K, BB), jnp.int32),
                       pltpu.VMEM((2, CR, CW), jnp.float32),
                       pltpu.VMEM_SHARED((nseg, CW), jnp.float32),
                       pltpu.SemaphoreType.DMA, pltpu.SemaphoreType.DMA,
                       pltpu.SemaphoreType.DMA, pltpu.SemaphoreType.DMA],
    )
    def k(eu_h, idx_h, z_h, out_h, idx_v, rows_v, acc_s, l0, l1, s0, s1):
        cid = lax.axis_index("c")
        sid = lax.axis_index("s")
        wid = sid * 2 + cid
        soff = pl.multiple_of(sid * SR, 8)
        wbase = pl.multiple_of(wid * KB, BB)
        lsems = (l0, l1)
        ssems = (s0, s1)
        pltpu.sync_copy(idx_h.at[wid], idx_v)
        for p in range(npass):
            pltpu.sync_copy(z_h.at[pl.ds(soff, SR)],
                            acc_s.at[pl.ds(soff, SR)])
            plsc.subcore_barrier()
            ld = [None] * NCH
            sd = [None] * NCH

            def fire_scatters(ch):
                ld[ch].wait()
                b = ch % 2
                ds = []
                for i in range(chunk_b):
                    j = ch * chunk_b + i
                    ds.append(pltpu.async_copy(
                        rows_v.at[b, pl.ds(i * BB, BB)],
                        acc_s.at[idx_v.at[j]], ssems[b], add=True))
                sd[ch] = ds

            for ch in range(NCH):
                b = ch % 2
                if ch >= 2:
                    for dsc in sd[ch - 2]:
                        dsc.wait()
                row0 = pl.multiple_of(wbase + ch * CR, BB)
                ld[ch] = pltpu.async_copy(
                    eu_h.at[pl.ds(row0, CR), pl.ds(p * CW, CW)],
                    rows_v.at[b], lsems[b])
                if ch >= 1:
                    fire_scatters(ch - 1)
            fire_scatters(NCH - 1)
            if NCH >= 2:
                for dsc in sd[NCH - 2]:
                    dsc.wait()
            for dsc in sd[NCH - 1]:
                dsc.wait()
            plsc.subcore_barrier()
            pltpu.sync_copy(
                acc_s.at[pl.ds(soff, SR)],
                out_h.at[cid, pl.ds(soff, SR), pl.ds(p * CW, CW)])
            plsc.subcore_barrier()

    return k(eu, idx2d, zeros)


def _pad_rows(x, n):
    return jnp.pad(x, ((0, n - x.shape[0]), (0, 0)))


def _pad_idx(idx, n):
    return jnp.pad(idx.astype(jnp.int32), (0, n - idx.shape[0]))


def kernel(grid_nfeat, mesh_nfeat, g2m_efeat, mesh_efeat, m2g_efeat,
           g2m_src, g2m_dst, mesh_src, mesh_dst, m2g_src, m2g_dst, params):
    p = params

    def W0(q):
        return q['l0']['W']

    def mlpw(q):
        return q['l0']['W'], q['l0']['b'], q['l1']['W'], q['l1']['b']

    def mlpn(q):
        return q['l0']['b'], q['l1']['W'], q['l1']['b'], q['g'], q['bn']

    x_grid = grid_nfeat[0].reshape(C_IN, NG).transpose(1, 0)
    x_grid = jnp.pad(x_grid, ((0, NGP - NG), (0, D - C_IN)))
    mesh_n = _pad_rows(mesh_nfeat, NMP)
    ge = _pad_rows(g2m_efeat, EGP)
    me = _pad_rows(mesh_efeat, EMP)
    de = _pad_rows(m2g_efeat, EGP)

    g2m_src_p = _pad_idx(g2m_src, EGP)
    m2g_src_p = _pad_idx(m2g_src, EGP)
    mesh_src_p = _pad_idx(mesh_src, EMP)
    g2m_dst_p = jnp.pad(g2m_dst.astype(jnp.int32), (0, EGP - E_G2M),
                        constant_values=NM)
    mesh_dst_p = jnp.pad(mesh_dst.astype(jnp.int32), (0, EMP - E_M),
                         constant_values=NM)
    m2g_dst_p = jnp.pad(m2g_dst.astype(jnp.int32), (0, EGP - E_M2G),
                        constant_values=NG)

    w_g2m = W0(p['g2m_edge_mlp'])
    w_g2m_e, w_g2m_s, w_g2m_d = w_g2m[:D], w_g2m[D:2 * D], w_g2m[2 * D:]
    w_m2g = W0(p['m2g_edge_mlp'])
    w_m2g_e, w_m2g_s, w_m2g_d = w_m2g[:D], w_m2g[D:2 * D], w_m2g[2 * D:]
    w_pe = [W0(p['proc%d_edge' % i]) for i in range(L)]
    w_pe_e = [w[:D] for w in w_pe]
    w_pe_s = [w[D:2 * D] for w in w_pe]
    w_pe_d = [w[2 * D:] for w in w_pe]

    gw0, gb0, gw1, gb1 = mlpw(p['grid_enc'])
    gw0 = jnp.pad(gw0, ((0, D - C_IN), (0, 0)))
    gfeat, t_g2m_src = _enc_wide(x_grid, gw0, gb0, gw1, gb1,
                                 p['grid_enc']['g'], p['grid_enc']['bn'],
                                 [w_g2m_s])
    mfeat, t_g2m_dst = _enc_cols(mesh_n, *mlpw(p['mesh_enc'])[0:1],
                                 *mlpn(p['mesh_enc']), tables=[w_g2m_d])
    (eterm_g2m,) = _enc_cols(ge, W0(p['g2m_edge_enc']),
                             *mlpn(p['g2m_edge_enc']), tables=[w_g2m_e],
                             emit_y=False)

    gs, gd = _sc_gather2(t_g2m_src, g2m_src_p, t_g2m_dst, g2m_dst_p)
    q = p['g2m_edge_mlp']
    (eu,) = _edge_mlp(eterm_g2m, gs, gd, *mlpn(q))
    aggs = _sc_segsum(eu, g2m_dst_p.reshape(NW, -1, BB), NMP, 1, 1)
    q = p['g2m_node_mlp']
    w0q = W0(q)
    mfeat, ts0, td0 = _node_mlp(mfeat, aggs, w0q[:D], w0q[D:], *mlpn(q),
                                tables=[w_pe_s[0], w_pe_d[0]])
    q = p['g2m_grid_mlp']
    gfeat, t_m2g_dst = _enc_wide_residual(gfeat, W0(q), *mlpn(q), [w_m2g_d])

    q = p['mesh_edge_enc']
    e_m, eterm = _enc_cols(me, W0(q), *mlpn(q), tables=[w_pe_e[0]],
                           emit_y=True)
    ts, td = ts0, td0
    for i in range(L):
        gs, gd = _sc_gather2(ts, mesh_src_p, td, mesh_dst_p)
        q = p['proc%d_edge' % i]
        if i < L - 1:
            eu, e_m, eterm = _edge_mlp(eterm, gs, gd, *mlpn(q), e_in=e_m,
                                       Wnext=w_pe_e[i + 1])
        else:
            (eu,) = _edge_mlp(eterm, gs, gd, *mlpn(q))
        aggs = _sc_segsum(eu, mesh_dst_p.reshape(NW, -1, BB), NMP, 1, 1)
        q = p['proc%d_node' % i]
        w0q = W0(q)
        if i < L - 1:
            tabs = [w_pe_s[i + 1], w_pe_d[i + 1]]
        else:
            tabs = [w_m2g_s]
        outs = _node_mlp(mfeat, aggs, w0q[:D], w0q[D:], *mlpn(q),
                         tables=tabs)
        mfeat = outs[0]
        if i < L - 1:
            ts, td = outs[1], outs[2]
        else:
            t_m2g_src = outs[1]

    q = p['m2g_edge_enc']
    (eterm_m2g,) = _enc_cols(de, W0(q), *mlpn(q), tables=[w_m2g_e],
                             emit_y=False)
    gs, gd = _sc_gather2(t_m2g_src, m2g_src_p, t_m2g_dst, m2g_dst_p)
    q = p['m2g_edge_mlp']
    (eu,) = _edge_mlp(eterm_m2g, gs, gd, *mlpn(q))
    RP = NGP // 8
    parts = []
    for pp in range(8):
        in_range = ((m2g_dst_p >= pp * RP) & (m2g_dst_p < (pp + 1) * RP))
        idxp = jnp.where(in_range, m2g_dst_p - pp * RP, RP)
        part = _sc_segsum(eu, idxp.reshape(NW, -1, BB), RP + 128, 1, 1)
        parts.append(part[:, :RP])
    aggs = jnp.concatenate(parts, axis=1)
    q = p['m2g_node_mlp']
    w0q = W0(q)
    dq = p['dec']
    dw1 = jnp.pad(dq['l1']['W'], ((0, 0), (0, D - C_OUT)))
    db1 = jnp.pad(dq['l1']['b'], (0, D - C_OUT))
    (out128,) = _node_mlp(gfeat, aggs, w0q[:D], w0q[D:], *mlpn(q),
                          dec=(dq['l0']['W'], dq['l0']['b'], dw1, db1))

    out = out128[:NG, :C_OUT]
    return out.transpose(1, 0).reshape(1, C_OUT, H, W)


def _enc_wide_residual(x, W0, b0, W1, b1, g, bn, tables):
    n = x.shape[0]
    nt = len(tables)

    def body(x_r, W0_r, b0_r, W1_r, b1_r, g_r, bn_r, *rest):
        t_refs = rest[:nt]
        o_refs = rest[nt:]
        xv = x_r[...]
        h = _silu(jnp.dot(xv, W0_r[...], preferred_element_type=_F32)
                  + b0_r[...])
        y = xv + _ln(jnp.dot(h, W1_r[...], preferred_element_type=_F32)
                     + b1_r[...], g_r[...], bn_r[...])
        o_refs[0][...] = y
        for k in range(nt):
            o_refs[k + 1][...] = jnp.dot(y, t_refs[k][...],
                                         preferred_element_type=_F32)

    return pl.pallas_call(
        body,
        grid=(n // BR,),
        in_specs=[_rows_bs(D), _full_bs((D, D)), _full_bs((1, D)),
                  _full_bs((D, D)), _full_bs((1, D)), _full_bs((1, D)),
                  _full_bs((1, D))] + [_full_bs((D, D))] * nt,
        out_specs=[_rows_bs(D)] * (1 + nt),
        out_shape=[jax.ShapeDtypeStruct((n, D), _F32)] * (1 + nt),
    )(x, W0, b0.reshape(1, D), W1, b1.reshape(1, D), g.reshape(1, D),
      bn.reshape(1, D), *tables)


# speedup vs baseline: 2.0998x; 1.0140x over previous
import functools

import jax
import jax.numpy as jnp
from jax import lax
from jax.experimental import pallas as pl
from jax.experimental.pallas import tpu as pltpu
from jax.experimental.pallas import tpu_sc as plsc

H, W, C_IN, C_OUT = 181, 360, 96, 48
NG = H * W
NM = 10242
E_G2M = 2 * NG
E_M = 8 * NM
E_M2G = 2 * NG
D = 128
L = 4

BR = 512
NGP = 65536
NMP = 10752
EGP = 131072
EMP = 86016

NW = 32
BB = 128

_F32 = jnp.float32
_BF16 = jnp.bfloat16


def _rows_bs(ncols):
    return pl.BlockSpec((BR, ncols), lambda i: (i, 0))


def _full_bs(shape):
    nd = len(shape)
    return pl.BlockSpec(shape, lambda i: (0,) * nd)


def _ln(h, g, bn):
    mu = jnp.mean(h, axis=-1, keepdims=True)
    var = jnp.mean((h - mu) ** 2, axis=-1, keepdims=True)
    return (h - mu) * jax.lax.rsqrt(var + 1e-5) * g + bn


def _silu(x):
    return x * jax.nn.sigmoid(x)


def _enc_wide(x, W0, b0, W1, b1, g, bn, tables):
    n = x.shape[0]
    nt = len(tables)

    def body(x_r, W0_r, b0_r, W1_r, b1_r, g_r, bn_r, *rest):
        t_refs = rest[:nt]
        o_refs = rest[nt:]
        h = _silu(jnp.dot(x_r[...], W0_r[...], preferred_element_type=_F32)
                  + b0_r[...])
        y = _ln(jnp.dot(h, W1_r[...], preferred_element_type=_F32) + b1_r[...],
                g_r[...], bn_r[...])
        o_refs[0][...] = y
        for k in range(nt):
            o_refs[k + 1][...] = jnp.dot(y, t_refs[k][...],
                                         preferred_element_type=_F32)

    out_shape = [jax.ShapeDtypeStruct((n, D), _F32)] * (1 + nt)
    return pl.pallas_call(
        body,
        grid=(n // BR,),
        in_specs=[_rows_bs(x.shape[1]), _full_bs(W0.shape), _full_bs((1, D)),
                  _full_bs((D, D)), _full_bs((1, D)), _full_bs((1, D)),
                  _full_bs((1, D))] + [_full_bs((D, D))] * nt,
        out_specs=[_rows_bs(D)] * (1 + nt),
        out_shape=out_shape,
    )(x, W0, b0.reshape(1, D), W1, b1.reshape(1, D), g.reshape(1, D),
      bn.reshape(1, D), *tables)


def _enc_cols(x, W0, b0, W1, b1, g, bn, tables, emit_y=True,
              tdt=_BF16):
    n, c = x.shape
    nt = len(tables)

    def body(x_r, W0_r, b0_r, W1_r, b1_r, g_r, bn_r, *rest):
        t_refs = rest[:nt]
        o_refs = rest[nt:]
        acc = b0_r[...]
        xv = x_r[...]
        w0 = W0_r[...]
        for k in range(c):
            acc = acc + xv[:, k:k + 1] * w0[k:k + 1, :]
        h = _silu(acc)
        y = _ln(jnp.dot(h, W1_r[...], preferred_element_type=_F32) + b1_r[...],
                g_r[...], bn_r[...])
        o = 0
        if emit_y:
            o_refs[0][...] = y
            o = 1
        for k in range(nt):
            o_refs[o + k][...] = jnp.dot(y, t_refs[k][...],
                                         preferred_element_type=_F32
                                         ).astype(tdt)

    n_out = (1 if emit_y else 0) + nt
    return pl.pallas_call(
        body,
        grid=(n // BR,),
        in_specs=[_rows_bs(c), _full_bs((c, D)), _full_bs((1, D)),
                  _full_bs((D, D)), _full_bs((1, D)), _full_bs((1, D)),
                  _full_bs((1, D))] + [_full_bs((D, D))] * nt,
        out_specs=[_rows_bs(D)] * n_out,
        out_shape=([jax.ShapeDtypeStruct((n, D), _F32)] * (1 if emit_y else 0)
                   + [jax.ShapeDtypeStruct((n, D), tdt)] * nt),
    )(x, W0, b0.reshape(1, D), W1, b1.reshape(1, D), g.reshape(1, D),
      bn.reshape(1, D), *tables)


def _edge_mlp(eterm, gs, gd, b0, W1, b1, g, bn, e_in=None, Wnext=None):
    n = eterm.shape[0]
    has_state = e_in is not None
    has_next = Wnext is not None

    def body(*refs):
        i = 0
        eterm_r = refs[i]; i += 1
        gs_r = refs[i]; i += 1
        gd_r = refs[i]; i += 1
        b0_r = refs[i]; i += 1
        W1_r = refs[i]; i += 1
        b1_r = refs[i]; i += 1
        g_r = refs[i]; i += 1
        bn_r = refs[i]; i += 1
        e_r = None
        Wn_r = None
        if has_state:
            e_r = refs[i]; i += 1
        if has_next:
            Wn_r = refs[i]; i += 1
        outs = refs[i:]
        h = _silu(eterm_r[...].astype(_F32) + gs_r[...].astype(_F32)
                  + gd_r[...].astype(_F32) + b0_r[...])
        eu = _ln(jnp.dot(h, W1_r[...], preferred_element_type=_F32)
                 + b1_r[...], g_r[...], bn_r[...])
        outs[0][...] = eu
        o = 1
        if has_state:
            e_new = e_r[...] + eu
            outs[o][...] = e_new
            o += 1
            if has_next:
                outs[o][...] = jnp.dot(e_new, Wn_r[...],
                                       preferred_element_type=_F32
                                       ).astype(_BF16)

    n_out = 1 + (1 if has_state else 0) + (1 if (has_state and has_next) else 0)
    in_arrs = [eterm, gs, gd, b0.reshape(1, D), W1, b1.reshape(1, D),
               g.reshape(1, D), bn.reshape(1, D)]
    in_specs = [_rows_bs(D), _rows_bs(D), _rows_bs(D), _full_bs((1, D)),
                _full_bs((D, D)), _full_bs((1, D)), _full_bs((1, D)),
                _full_bs((1, D))]
    if has_state:
        in_arrs.append(e_in)
        in_specs.append(_rows_bs(D))
    if has_next:
        in_arrs.append(Wnext)
        in_specs.append(_full_bs((D, D)))
    out_shape = [jax.ShapeDtypeStruct((n, D), _F32)] * min(n_out, 2)
    if n_out == 3:
        out_shape.append(jax.ShapeDtypeStruct((n, D), _BF16))
    return pl.pallas_call(
        body,
        grid=(n // BR,),
        in_specs=in_specs,
        out_specs=[_rows_bs(D)] * n_out,
        out_shape=out_shape,
    )(*in_arrs)


def _node_mlp(x, aggs, W0x, W0a, b0, W1, b1, g, bn, tables=(), dec=None):
    n = x.shape[0]
    ns = aggs.shape[0]
    nt = len(tables)
    has_dec = dec is not None

    def body(*refs):
        i = 0
        x_r = refs[i]; i += 1
        a_r = refs[i]; i += 1
        W0x_r = refs[i]; i += 1
        W0a_r = refs[i]; i += 1
        b0_r = refs[i]; i += 1
        W1_r = refs[i]; i += 1
        b1_r = refs[i]; i += 1
        g_r = refs[i]; i += 1
        bn_r = refs[i]; i += 1
        t_refs = refs[i:i + nt]; i += nt
        if has_dec:
            Wd0_r = refs[i]; i += 1
            bd0_r = refs[i]; i += 1
            Wd1_r = refs[i]; i += 1
            bd1_r = refs[i]; i += 1
        outs = refs[i:]
        av = a_r[...]
        agg = av[0]
        for k in range(1, ns):
            agg = agg + av[k]
        xv = x_r[...]
        h = _silu(jnp.dot(xv, W0x_r[...], preferred_element_type=_F32)
                  + jnp.dot(agg, W0a_r[...], preferred_element_type=_F32)
                  + b0_r[...])
        y = _ln(jnp.dot(h, W1_r[...], preferred_element_type=_F32)
                + b1_r[...], g_r[...], bn_r[...])
        x_new = xv + y
        o = 0
        if has_dec:
            hd = _silu(jnp.dot(x_new, Wd0_r[...],
                               preferred_element_type=_F32) + bd0_r[...])
            outs[0][...] = jnp.dot(hd, Wd1_r[...],
                                   preferred_element_type=_F32) + bd1_r[...]
            o = 1
        else:
            outs[0][...] = x_new
            o = 1
        for k in range(nt):
            outs[o + k][...] = jnp.dot(x_new, t_refs[k][...],
                                       preferred_element_type=_F32)

    in_arrs = [x, aggs] + [W0x, W0a, b0.reshape(1, D), W1,
                           b1.reshape(1, D), g.reshape(1, D),
                           bn.reshape(1, D)] + list(tables)
    in_specs = ([_rows_bs(D),
                 pl.BlockSpec((ns, BR, D), lambda i: (0, i, 0))]
                + [_full_bs((D, D)), _full_bs((D, D)), _full_bs((1, D)),
                   _full_bs((D, D)), _full_bs((1, D)), _full_bs((1, D)),
                   _full_bs((1, D))] + [_full_bs((D, D))] * nt)
    if has_dec:
        Wd0, bd0, Wd1, bd1 = dec
        in_arrs += [Wd0, bd0.reshape(1, D), Wd1, bd1.reshape(1, D)]
        in_specs += [_full_bs((D, D)), _full_bs((1, D)), _full_bs((D, D)),
                     _full_bs((1, D))]
    n_out = 1 + nt
    return pl.pallas_call(
        body,
        grid=(n // BR,),
        in_specs=in_specs,
        out_specs=[_rows_bs(D)] * n_out,
        out_shape=[jax.ShapeDtypeStruct((n, D), _F32)] * n_out,
    )(*in_arrs)


def _sc_gather2(tabA, idxA, tabB, idxB):
    EP = idxA.shape[0]
    KB = EP // NW
    K = KB // BB
    dt = tabA.dtype
    mesh = plsc.VectorSubcoreMesh(core_axis_name="c", subcore_axis_name="s")

    @functools.partial(
        pl.kernel,
        out_type=[jax.ShapeDtypeStruct((EP, D), dt),
                  jax.ShapeDtypeStruct((EP, D), dt)],
        mesh=mesh,
        scratch_types=[pltpu.VMEM((KB,), jnp.int32),
                       pltpu.VMEM((4, BB, D), dt),
                       pltpu.SemaphoreType.DMA, pltpu.SemaphoreType.DMA,
                       pltpu.SemaphoreType.DMA, pltpu.SemaphoreType.DMA,
                       pltpu.SemaphoreType.DMA, pltpu.SemaphoreType.DMA,
                       pltpu.SemaphoreType.DMA, pltpu.SemaphoreType.DMA],
    )
    def k(tabA_h, idxA_h, tabB_h, idxB_h, outA_h, outB_h,
          idx_v, rows_v, *sems):
        wid = lax.axis_index("s") * 2 + lax.axis_index("c")
        base = pl.multiple_of(wid * KB, BB)
        gsems = sems[0:4]
        wsems = sems[4:8]
        for tab_h, idx_h, out_h in ((tabA_h, idxA_h, outA_h),
                                    (tabB_h, idxB_h, outB_h)):
            pltpu.sync_copy(idx_h.at[pl.ds(base, KB)], idx_v)
            gd = [None] * K
            wd = [None] * K

            def write(j):
                gd[j].wait()
                wd[j] = pltpu.async_copy(
                    rows_v.at[j % 4], out_h.at[pl.ds(base + j * BB, BB)],
                    wsems[j % 4])

            for j in range(K):
                b = j % 4
                if j >= 4:
                    wd[j - 4].wait()
                gd[j] = pltpu.async_copy(
                    tab_h.at[idx_v.at[pl.ds(j * BB, BB)]], rows_v.at[b],
                    gsems[b])
                if j >= 2:
                    write(j - 2)
            write(K - 2)
            write(K - 1)
            for j in range(max(0, K - 4), K):
                wd[j].wait()

    return k(tabA, idxA, tabB, idxB)


def _sc_segsum(eu, idx2d, nseg, npass, chunk_b):
    EP = eu.shape[0]
    KB = EP // NW
    K = KB // BB
    NCH = KB // (chunk_b * BB)
    CW = D // npass
    SR = nseg // 16
    CR = chunk_b * BB
    mesh = plsc.VectorSubcoreMesh(core_axis_name="c", subcore_axis_name="s")
    zeros = jnp.zeros((nseg, CW), jnp.float32)

    @functools.partial(
        pl.kernel,
        out_type=jax.ShapeDtypeStruct((2, nseg, D), jnp.float32),
        mesh=mesh,
        scratch_types=[pltpu.VMEM((K, BB), jnp.int32),
                       pltpu.VMEM((2, CR, CW), jnp.float32),
                       pltpu.VMEM_SHARED((nseg, CW), jnp.float32),
                       pltpu.SemaphoreType.DMA, pltpu.SemaphoreType.DMA,
                       pltpu.SemaphoreType.DMA, pltpu.SemaphoreType.DMA],
    )
    def k(eu_h, idx_h, z_h, out_h, idx_v, rows_v, acc_s, l0, l1, s0, s1):
        cid = lax.axis_index("c")
        sid = lax.axis_index("s")
        wid = sid * 2 + cid
        soff = pl.multiple_of(sid * SR, 8)
        wbase = pl.multiple_of(wid * KB, BB)
        lsems = (l0, l1)
        ssems = (s0, s1)
        pltpu.sync_copy(idx_h.at[wid], idx_v)
        for p in range(npass):
            pltpu.sync_copy(z_h.at[pl.ds(soff, SR)],
                            acc_s.at[pl.ds(soff, SR)])
            plsc.subcore_barrier()
            ld = [None] * NCH
            sd = [None] * NCH

            def fire_scatters(ch):
                ld[ch].wait()
                b = ch % 2
                ds = []
                for i in range(chunk_b):
                    j = ch * chunk_b + i
                    ds.append(pltpu.async_copy(
                        rows_v.at[b, pl.ds(i * BB, BB)],
                        acc_s.at[idx_v.at[j]], ssems[b], add=True))
                sd[ch] = ds

            for ch in range(NCH):
                b = ch % 2
                if ch >= 2:
                    for dsc in sd[ch - 2]:
                        dsc.wait()
                row0 = pl.multiple_of(wbase + ch * CR, BB)
                ld[ch] = pltpu.async_copy(
                    eu_h.at[pl.ds(row0, CR), pl.ds(p * CW, CW)],
                    rows_v.at[b], lsems[b])
                if ch >= 1:
                    fire_scatters(ch - 1)
            fire_scatters(NCH - 1)
            if NCH >= 2:
                for dsc in sd[NCH - 2]:
                    dsc.wait()
            for dsc in sd[NCH - 1]:
                dsc.wait()
            plsc.subcore_barrier()
            pltpu.sync_copy(
                acc_s.at[pl.ds(soff, SR)],
                out_h.at[cid, pl.ds(soff, SR), pl.ds(p * CW, CW)])
            plsc.subcore_barrier()

    return k(eu, idx2d, zeros)


def _pad_rows(x, n):
    return jnp.pad(x, ((0, n - x.shape[0]), (0, 0)))


def _pad_idx(idx, n):
    return jnp.pad(idx.astype(jnp.int32), (0, n - idx.shape[0]))


def kernel(grid_nfeat, mesh_nfeat, g2m_efeat, mesh_efeat, m2g_efeat,
           g2m_src, g2m_dst, mesh_src, mesh_dst, m2g_src, m2g_dst, params):
    p = params

    def W0(q):
        return q['l0']['W']

    def mlpw(q):
        return q['l0']['W'], q['l0']['b'], q['l1']['W'], q['l1']['b']

    def mlpn(q):
        return q['l0']['b'], q['l1']['W'], q['l1']['b'], q['g'], q['bn']

    x_grid = grid_nfeat[0].reshape(C_IN, NG).transpose(1, 0)
    x_grid = jnp.pad(x_grid, ((0, NGP - NG), (0, D - C_IN)))
    mesh_n = _pad_rows(mesh_nfeat, NMP)
    ge = _pad_rows(g2m_efeat, EGP)
    me = _pad_rows(mesh_efeat, EMP)
    de = _pad_rows(m2g_efeat, EGP)

    g2m_src_p = _pad_idx(g2m_src, EGP)
    m2g_src_p = _pad_idx(m2g_src, EGP)
    mesh_src_p = _pad_idx(mesh_src, EMP)
    g2m_dst_p = jnp.pad(g2m_dst.astype(jnp.int32), (0, EGP - E_G2M),
                        constant_values=NM)
    mesh_dst_p = jnp.pad(mesh_dst.astype(jnp.int32), (0, EMP - E_M),
                         constant_values=NM)
    m2g_dst_p = jnp.pad(m2g_dst.astype(jnp.int32), (0, EGP - E_M2G),
                        constant_values=NG)

    w_g2m = W0(p['g2m_edge_mlp'])
    w_g2m_e, w_g2m_s, w_g2m_d = w_g2m[:D], w_g2m[D:2 * D], w_g2m[2 * D:]
    w_m2g = W0(p['m2g_edge_mlp'])
    w_m2g_e, w_m2g_s, w_m2g_d = w_m2g[:D], w_m2g[D:2 * D], w_m2g[2 * D:]
    w_pe = [W0(p['proc%d_edge' % i]) for i in range(L)]
    w_pe_e = [w[:D] for w in w_pe]
    w_pe_s = [w[D:2 * D] for w in w_pe]
    w_pe_d = [w[2 * D:] for w in w_pe]

    gw0, gb0, gw1, gb1 = mlpw(p['grid_enc'])
    gw0 = jnp.pad(gw0, ((0, D - C_IN), (0, 0)))
    gfeat, t_g2m_src = _enc_wide(x_grid, gw0, gb0, gw1, gb1,
                                 p['grid_enc']['g'], p['grid_enc']['bn'],
                                 [w_g2m_s])
    mfeat, t_g2m_dst = _enc_cols(mesh_n, *mlpw(p['mesh_enc'])[0:1],
                                 *mlpn(p['mesh_enc']), tables=[w_g2m_d],
                                 tdt=_F32)
    (eterm_g2m,) = _enc_cols(ge, W0(p['g2m_edge_enc']),
                             *mlpn(p['g2m_edge_enc']), tables=[w_g2m_e],
                             emit_y=False)

    gs, gd = _sc_gather2(t_g2m_src, g2m_src_p, t_g2m_dst, g2m_dst_p)
    q = p['g2m_edge_mlp']
    (eu,) = _edge_mlp(eterm_g2m, gs, gd, *mlpn(q))
    aggs = _sc_segsum(eu, g2m_dst_p.reshape(NW, -1, BB), NMP, 1, 1)
    q = p['g2m_node_mlp']
    w0q = W0(q)
    mfeat, ts0, td0 = _node_mlp(mfeat, aggs, w0q[:D], w0q[D:], *mlpn(q),
                                tables=[w_pe_s[0], w_pe_d[0]])
    q = p['g2m_grid_mlp']
    gfeat, t_m2g_dst = _enc_wide_residual(gfeat, W0(q), *mlpn(q), [w_m2g_d])

    q = p['mesh_edge_enc']
    e_m, eterm = _enc_cols(me, W0(q), *mlpn(q), tables=[w_pe_e[0]],
                           emit_y=True)
    ts, td = ts0, td0
    for i in range(L):
        gs, gd = _sc_gather2(ts, mesh_src_p, td, mesh_dst_p)
        q = p['proc%d_edge' % i]
        if i < L - 1:
            eu, e_m, eterm = _edge_mlp(eterm, gs, gd, *mlpn(q), e_in=e_m,
                                       Wnext=w_pe_e[i + 1])
        else:
            (eu,) = _edge_mlp(eterm, gs, gd, *mlpn(q))
        aggs = _sc_segsum(eu, mesh_dst_p.reshape(NW, -1, BB), NMP, 1, 1)
        q = p['proc%d_node' % i]
        w0q = W0(q)
        if i < L - 1:
            tabs = [w_pe_s[i + 1], w_pe_d[i + 1]]
        else:
            tabs = [w_m2g_s]
        outs = _node_mlp(mfeat, aggs, w0q[:D], w0q[D:], *mlpn(q),
                         tables=tabs)
        mfeat = outs[0]
        if i < L - 1:
            ts, td = outs[1], outs[2]
        else:
            t_m2g_src = outs[1]

    q = p['m2g_edge_enc']
    (eterm_m2g,) = _enc_cols(de, W0(q), *mlpn(q), tables=[w_m2g_e],
                             emit_y=False)
    gs, gd = _sc_gather2(t_m2g_src, m2g_src_p, t_m2g_dst, m2g_dst_p)
    q = p['m2g_edge_mlp']
    (eu,) = _edge_mlp(eterm_m2g, gs, gd, *mlpn(q))
    RP = NGP // 8
    parts = []
    for pp in range(8):
        in_range = ((m2g_dst_p >= pp * RP) & (m2g_dst_p < (pp + 1) * RP))
        idxp = jnp.where(in_range, m2g_dst_p - pp * RP, RP)
        part = _sc_segsum(eu, idxp.reshape(NW, -1, BB), RP + 128, 1, 1)
        parts.append(part[:, :RP])
    aggs = jnp.concatenate(parts, axis=1)
    q = p['m2g_node_mlp']
    w0q = W0(q)
    dq = p['dec']
    dw1 = jnp.pad(dq['l1']['W'], ((0, 0), (0, D - C_OUT)))
    db1 = jnp.pad(dq['l1']['b'], (0, D - C_OUT))
    (out128,) = _node_mlp(gfeat, aggs, w0q[:D], w0q[D:], *mlpn(q),
                          dec=(dq['l0']['W'], dq['l0']['b'], dw1, db1))

    out = out128[:NG, :C_OUT]
    return out.transpose(1, 0).reshape(1, C_OUT, H, W)


def _enc_wide_residual(x, W0, b0, W1, b1, g, bn, tables):
    n = x.shape[0]
    nt = len(tables)

    def body(x_r, W0_r, b0_r, W1_r, b1_r, g_r, bn_r, *rest):
        t_refs = rest[:nt]
        o_refs = rest[nt:]
        xv = x_r[...]
        h = _silu(jnp.dot(xv, W0_r[...], preferred_element_type=_F32)
                  + b0_r[...])
        y = xv + _ln(jnp.dot(h, W1_r[...], preferred_element_type=_F32)
                     + b1_r[...], g_r[...], bn_r[...])
        o_refs[0][...] = y
        for k in range(nt):
            o_refs[k + 1][...] = jnp.dot(y, t_refs[k][...],
                                         preferred_element_type=_F32)

    return pl.pallas_call(
        body,
        grid=(n // BR,),
        in_specs=[_rows_bs(D), _full_bs((D, D)), _full_bs((1, D)),
                  _full_bs((D, D)), _full_bs((1, D)), _full_bs((1, D)),
                  _full_bs((1, D))] + [_full_bs((D, D))] * nt,
        out_specs=[_rows_bs(D)] * (1 + nt),
        out_shape=[jax.ShapeDtypeStruct((n, D), _F32)] * (1 + nt),
    )(x, W0, b0.reshape(1, D), W1, b1.reshape(1, D), g.reshape(1, D),
      bn.reshape(1, D), *tables)
